# baseline (speedup 1.0000x reference)
import functools

import jax
import jax.numpy as jnp
from jax import lax
from jax.experimental import pallas as pl
from jax.experimental.pallas import tpu as pltpu
from jax.experimental.pallas import tpu_sc as plsc

N = 10000
E = 160000
NC = 2
NS = 16
L = 16
NPAD = 10240
EPAD = 163840
CH = 128
ROWS_PER_TEC = NPAD // NS
CHUNKS_TOTAL = EPAD // CH
CHUNKS_PER_TEC = CHUNKS_TOTAL // NS
EPB = EPAD // (NC * NS)

@functools.lru_cache(maxsize=None)
def _mesh():
  return plsc.VectorSubcoreMesh(
      core_axis_name="c", subcore_axis_name="s", num_cores=NC, num_subcores=NS)


def _deg_body(dst_hbm, out_hbm, dstv, histv, tmpv, colv, shared):
  c = lax.axis_index("c")
  t = lax.axis_index("s")
  w = c * NS + t
  ones = jnp.full((L,), 1.0, jnp.float32)
  zeros = jnp.zeros((L,), jnp.float32)

  def zero_hist(i, carry):
    histv[pl.ds(i * L, L)] = zeros
    return carry
  lax.fori_loop(0, NPAD // L, zero_hist, 0)

  pltpu.sync_copy(dst_hbm.at[pl.ds(w * EPB, EPB)], dstv)

  def hist_step(i, carry):
    idx = dstv[pl.ds(i * L, L)]
    plsc.addupdate_scatter(histv, [idx], ones)
    return carry
  lax.fori_loop(0, EPB // L, hist_step, 0)

  pltpu.sync_copy(histv, shared.at[t])
  plsc.subcore_barrier()

  cb = t * ROWS_PER_TEC
  for k in range(NS):
    pltpu.sync_copy(shared.at[k, pl.ds(cb, ROWS_PER_TEC)], tmpv.at[k])

  def col_sum(i, carry):
    acc = tmpv[0, pl.ds(i * L, L)]
    for k in range(1, NS):
      acc = acc + tmpv[k, pl.ds(i * L, L)]
    colv[pl.ds(i * L, L)] = acc
    return carry
  lax.fori_loop(0, ROWS_PER_TEC // L, col_sum, 0)

  pltpu.sync_copy(colv, out_hbm.at[pl.ds(c * NPAD + cb, ROWS_PER_TEC)])


@functools.lru_cache(maxsize=None)
def _deg_kernel():
  return pl.kernel(
    _deg_body,
    out_type=jax.ShapeDtypeStruct((NC * NPAD,), jnp.float32),
    mesh=_mesh(),
    scratch_types=[
        pltpu.VMEM((EPB,), jnp.int32),
        pltpu.VMEM((NPAD,), jnp.float32),
        pltpu.VMEM((NS, ROWS_PER_TEC), jnp.float32),
        pltpu.VMEM((ROWS_PER_TEC,), jnp.float32),
        pltpu.VMEM_SHARED((NS, NPAD), jnp.float32),
    ],
    compiler_params=pltpu.CompilerParams(needs_layout_passes=False),
  )


CB = 64
NCH = EPAD // (NS * CB)


@functools.lru_cache(maxsize=None)
def _make_prop(S):
  s_half = S // NC
  epb = NCH * CB
  nhalf = NCH // 2

  def body(h_hbm, src_hbm, dst2_hbm, zeros_hbm, out_hbm,
           srcf, dstb, bf_v, fbuf, acc_sh, sem_ga, sem_gb, sem_sa, sem_sb):
    c = lax.axis_index("c")
    t = lax.axis_index("s")
    my_rows = t * ROWS_PER_TEC
    gsems = (sem_ga, sem_gb)
    ssems = (sem_sa, sem_sb)

    pltpu.sync_copy(src_hbm.at[pl.ds(t * epb, epb)], srcf)
    pltpu.sync_copy(dst2_hbm.at[pl.ds(t * epb, epb)], dstb)

    def adjust(delta):
      dv = jnp.full((L,), delta, jnp.int32)
      def go(i, carry):
        srcf[pl.ds(i * L, L)] = srcf[pl.ds(i * L, L)] + dv
        return carry
      lax.fori_loop(0, epb // L, go, 0)

    def gidx(j):
      return srcf.at[pl.ds(j * CB, CB)]

    def bfb(p):
      return bf_v.at[pl.ds(p * CB, CB)]

    def fb(p):
      return fbuf.at[pl.ds(p * CB, CB)]

    def issue_g(ch, p):
      pltpu.async_copy(h_hbm.at[gidx(ch)], bfb(p), gsems[p])

    def wait_g(p):
      pltpu.make_async_copy(h_hbm.at[gidx(0)], bfb(p), gsems[p]).wait()

    def issue_s(ch, p):
      pltpu.async_copy(fb(p), acc_sh.at[dstb.at[pl.ds(ch * CB, CB)]],
                       ssems[p], add=True)

    def wait_s(p):
      pltpu.make_async_copy(fb(p), acc_sh.at[dstb.at[pl.ds(0, CB)]],
                            ssems[p]).wait()

    def convert(p):
      mask = jnp.full((L,), -65536, jnp.int32)
      sh = jnp.full((L,), 16, jnp.int32)
      def row(r4, carry):
        for u in range(4):
          r = 4 * r4 + u
          for g in range(4):
            y = bf_v[p * CB + r, pl.ds(g * L, L)]
            fbuf[p * CB + r, pl.ds(g * 2 * L, L)] = plsc.bitcast(
                jnp.left_shift(y, sh), jnp.float32)
            fbuf[p * CB + r, pl.ds(g * 2 * L + L, L)] = plsc.bitcast(
                y & mask, jnp.float32)
        return carry
      lax.fori_loop(0, CB // 4, row, 0)

    adjust(c * (s_half * N))
    for si in range(s_half):
      if si:
        adjust(N)
      pltpu.sync_copy(zeros_hbm, acc_sh.at[pl.ds(my_rows, ROWS_PER_TEC)])
      plsc.subcore_barrier()

      issue_g(0, 0)

      def pipe(j, carry):
        j0 = 2 * j
        issue_g(j0 + 1, 1)
        wait_g(0)
        @pl.when(j > 0)
        def _():
          wait_s(0)
        convert(0)
        issue_s(j0, 0)
        @pl.when(j < nhalf - 1)
        def _():
          issue_g(j0 + 2, 0)
        wait_g(1)
        @pl.when(j > 0)
        def _():
          wait_s(1)
        convert(1)
        issue_s(j0 + 1, 1)
        return carry
      lax.fori_loop(0, nhalf, pipe, 0)

      wait_s(0)
      wait_s(1)
      plsc.subcore_barrier()
      ob = (c * s_half + si) * NPAD + my_rows
      pltpu.sync_copy(acc_sh.at[pl.ds(my_rows, ROWS_PER_TEC)],
                      out_hbm.at[pl.ds(ob, ROWS_PER_TEC)])

  return pl.kernel(
      body,
      out_type=jax.ShapeDtypeStruct((S * NPAD, 128), jnp.float32),
      mesh=_mesh(),
      scratch_types=[
          pltpu.VMEM((epb,), jnp.int32),
          pltpu.VMEM((epb,), jnp.int32),
          pltpu.VMEM((2 * CB, 64), jnp.int32),
          pltpu.VMEM((2 * CB, 128), jnp.float32),
          pltpu.VMEM_SHARED((NPAD, 128), jnp.float32),
          pltpu.SemaphoreType.DMA,
          pltpu.SemaphoreType.DMA,
          pltpu.SemaphoreType.DMA,
          pltpu.SemaphoreType.DMA,
      ],
      compiler_params=pltpu.CompilerParams(needs_layout_passes=False,
                                           use_tc_tiling_on_sc=False),
  )


_BN = 1000
_GRID = N // _BN


def _pack128(h128):
  words = []
  for i in range(4):
    lo = h128[:, 32 * i:32 * i + 16]
    hi = h128[:, 32 * i + 16:32 * i + 32]
    lo16 = lax.bitcast_convert_type(lo.astype(jnp.bfloat16),
                                    jnp.uint16).astype(jnp.int32)
    hi16 = lax.bitcast_convert_type(hi.astype(jnp.bfloat16),
                                    jnp.uint16).astype(jnp.int32)
    words.append((hi16 << 16) | lo16)
  return jnp.concatenate(words, axis=1)


def _mm1_body(x_ref, w_ref, d0_ref, d1_ref, hsl_ref, hslb_ref, dinv_ref):
  dinv = lax.rsqrt(1.0 + d0_ref[...] + d1_ref[...])
  dinv_ref[...] = dinv
  h = jnp.dot(x_ref[...], w_ref[...], preferred_element_type=jnp.float32)
  h = h * dinv
  for s in range(8):
    hsl_ref[s] = h[:, 128 * s:128 * (s + 1)]
    hslb_ref[s] = _pack128(h[:, 128 * s:128 * (s + 1)])


def _mm1_call(x, wcat, d0, d1):
  return pl.pallas_call(
      _mm1_body,
      grid=(_GRID,),
      in_specs=[
          pl.BlockSpec((_BN, 256), lambda i: (i, 0)),
          pl.BlockSpec((256, 1024), lambda i: (0, 0)),
          pl.BlockSpec((_BN, 1), lambda i: (i, 0)),
          pl.BlockSpec((_BN, 1), lambda i: (i, 0)),
      ],
      out_specs=[
          pl.BlockSpec((8, _BN, 128), lambda i: (0, i, 0)),
          pl.BlockSpec((8, _BN, 64), lambda i: (0, i, 0)),
          pl.BlockSpec((_BN, 1), lambda i: (i, 0)),
      ],
      out_shape=[
          jax.ShapeDtypeStruct((8, N, 128), jnp.float32),
          jax.ShapeDtypeStruct((8, N, 64), jnp.int32),
          jax.ShapeDtypeStruct((N, 1), jnp.float32),
      ],
  )(x, wcat, d0, d1)


def _ef_body(acc_ref, hsl_ref, dinv_ref, b_ref, w_ref,
             x1_ref, h1_ref, hsl2_ref, hslb2_ref):
  dinv = dinv_ref[...]
  zs = []
  for s in range(8):
    z = dinv * (acc_ref[s] + hsl_ref[s]) + b_ref[s]
    zs.append(jnp.maximum(z, 0.0))
  x1 = jnp.concatenate(zs[:4], axis=1)
  h1 = jnp.concatenate(zs[4:], axis=1)
  x1_ref[...] = x1
  h1_ref[...] = h1
  hh = jnp.dot(h1, w_ref[...], preferred_element_type=jnp.float32) * dinv
  for s in range(4):
    hsl2_ref[s] = hh[:, 128 * s:128 * (s + 1)]
    hslb2_ref[s] = _pack128(hh[:, 128 * s:128 * (s + 1)])


def _ef_call(acc1, hsl1, dinv, bcat, w2m):
  return pl.pallas_call(
      _ef_body,
      grid=(_GRID,),
      in_specs=[
          pl.BlockSpec((8, _BN, 128), lambda i: (0, i, 0)),
          pl.BlockSpec((8, _BN, 128), lambda i: (0, i, 0)),
          pl.BlockSpec((_BN, 1), lambda i: (i, 0)),
          pl.BlockSpec((8, 1, 128), lambda i: (0, 0, 0)),
          pl.BlockSpec((512, 512), lambda i: (0, 0)),
      ],
      out_specs=[
          pl.BlockSpec((_BN, 512), lambda i: (i, 0)),
          pl.BlockSpec((_BN, 512), lambda i: (i, 0)),
          pl.BlockSpec((4, _BN, 128), lambda i: (0, i, 0)),
          pl.BlockSpec((4, _BN, 64), lambda i: (0, i, 0)),
      ],
      out_shape=[
          jax.ShapeDtypeStruct((N, 512), jnp.float32),
          jax.ShapeDtypeStruct((N, 512), jnp.float32),
          jax.ShapeDtypeStruct((4, N, 128), jnp.float32),
          jax.ShapeDtypeStruct((4, N, 64), jnp.int32),
      ],
  )(acc1, hsl1, dinv, bcat, w2m)


def _gh_body(acc_ref, hsl_ref, dinv_ref, b_ref, x1_ref, wa_ref, wb_ref,
             h2_ref, hsl3_ref, hslb3_ref):
  dinv = dinv_ref[...]
  zs = []
  for s in range(4):
    z = dinv * (acc_ref[s] + hsl_ref[s]) + b_ref[s]
    zs.append(jnp.maximum(z, 0.0))
  h2 = jnp.concatenate(zs, axis=1)
  h2_ref[...] = h2
  y = (jnp.dot(x1_ref[...], wa_ref[...], preferred_element_type=jnp.float32)
       + jnp.dot(h2, wb_ref[...], preferred_element_type=jnp.float32)) * dinv
  for s in range(2):
    hsl3_ref[s] = y[:, 128 * s:128 * (s + 1)]
    hslb3_ref[s] = _pack128(y[:, 128 * s:128 * (s + 1)])


def _gh_call(acc2, hsl2, dinv, b2, x1, wa, wb):
  return pl.pallas_call(
      _gh_body,
      grid=(_GRID,),
      in_specs=[
          pl.BlockSpec((4, _BN, 128), lambda i: (0, i, 0)),
          pl.BlockSpec((4, _BN, 128), lambda i: (0, i, 0)),
          pl.BlockSpec((_BN, 1), lambda i: (i, 0)),
          pl.BlockSpec((4, 1, 128), lambda i: (0, 0, 0)),
          pl.BlockSpec((_BN, 512), lambda i: (i, 0)),
          pl.BlockSpec((512, 256), lambda i: (0, 0)),
          pl.BlockSpec((512, 256), lambda i: (0, 0)),
      ],
      out_specs=[
          pl.BlockSpec((_BN, 512), lambda i: (i, 0)),
          pl.BlockSpec((2, _BN, 128), lambda i: (0, i, 0)),
          pl.BlockSpec((2, _BN, 64), lambda i: (0, i, 0)),
      ],
      out_shape=[
          jax.ShapeDtypeStruct((N, 512), jnp.float32),
          jax.ShapeDtypeStruct((2, N, 128), jnp.float32),
          jax.ShapeDtypeStruct((2, N, 64), jnp.int32),
      ],
  )(acc2, hsl2, dinv, b2, x1, wa, wb)


def _ep3_body(acc_ref, hsl_ref, dinv_ref, b_ref, out_ref):
  dinv = dinv_ref[...]
  for s in range(2):
    out_ref[:, 128 * s:128 * (s + 1)] = (
        dinv * (acc_ref[s] + hsl_ref[s]) + b_ref[s])


def _ep3_call(acc3, hsl3, dinv, bo):
  return pl.pallas_call(
      _ep3_body,
      grid=(_GRID,),
      in_specs=[
          pl.BlockSpec((2, _BN, 128), lambda i: (0, i, 0)),
          pl.BlockSpec((2, _BN, 128), lambda i: (0, i, 0)),
          pl.BlockSpec((_BN, 1), lambda i: (i, 0)),
          pl.BlockSpec((2, 1, 128), lambda i: (0, 0, 0)),
      ],
      out_specs=pl.BlockSpec((_BN, 256), lambda i: (i, 0)),
      out_shape=jax.ShapeDtypeStruct((N, 256), jnp.float32),
  )(acc3, hsl3, dinv, bo)


@jax.jit
def _run(x, edge_index, W1h, b1h, W2h, b2h, W2m, b2m, Wout, bout):
  src = edge_index[0]
  dst = edge_index[1]
  srcp = jnp.concatenate([src, jnp.zeros((EPAD - E,), jnp.int32)])
  dstp = jnp.concatenate([dst, jnp.full((EPAD - E,), NPAD - 1, jnp.int32)])
  dst2 = dstp
  zeros128 = jnp.zeros((ROWS_PER_TEC, 128), jnp.float32)

  degp = _deg_kernel()(dstp)
  d0 = degp[:N].reshape(N, 1)
  d1 = degp[NPAD:NPAD + N].reshape(N, 1)

  wcat = jnp.concatenate([W1h, W2h], axis=1)
  bcat = jnp.concatenate([b1h, b2h]).reshape(8, 1, 128)

  hsl1, hslb1, dinv = _mm1_call(x, wcat, d0, d1)
  acc1 = _make_prop(8)(hslb1.reshape(8 * N, 64), srcp, dst2, zeros128)
  x1, h1, hsl2, hslb2 = _ef_call(acc1.reshape(8, NPAD, 128), hsl1, dinv,
                                 bcat, W2m)

  acc2 = _make_prop(4)(hslb2.reshape(4 * N, 64), srcp, dst2, zeros128)
  h2, hsl3, hslb3 = _gh_call(acc2.reshape(4, NPAD, 128), hsl2, dinv,
                             b2m.reshape(4, 1, 128), x1, Wout[:512],
                             Wout[512:])

  acc3 = _make_prop(2)(hslb3.reshape(2 * N, 64), srcp, dst2, zeros128)
  out = _ep3_call(acc3.reshape(2, NPAD, 128), hsl3, dinv,
                  bout.reshape(2, 1, 128))
  return out, x1, h1, h2


def kernel(x, edge_index, percent, ricci_curvature,
           W1h, b1h, W2h, b2h, W2m, b2m, Wout, bout):
  del percent, ricci_curvature
  return _run(x, edge_index, W1h, b1h, W2h, b2h, W2m, b2m, Wout, bout)


# speedup vs baseline: 1.0027x; 1.0027x over previous
import functools

import jax
import jax.numpy as jnp
from jax import lax
from jax.experimental import pallas as pl
from jax.experimental.pallas import tpu as pltpu
from jax.experimental.pallas import tpu_sc as plsc

N = 10000
E = 160000
NC = 2
NS = 16
L = 16
NPAD = 10240
EPAD = 163840
CH = 128
ROWS_PER_TEC = NPAD // NS
CHUNKS_TOTAL = EPAD // CH
CHUNKS_PER_TEC = CHUNKS_TOTAL // NS
EPB = EPAD // (NC * NS)

@functools.lru_cache(maxsize=None)
def _mesh():
  return plsc.VectorSubcoreMesh(
      core_axis_name="c", subcore_axis_name="s", num_cores=NC, num_subcores=NS)


def _deg_body(dst_hbm, out_hbm, dstv, histv, tmpv, colv, shared):
  c = lax.axis_index("c")
  t = lax.axis_index("s")
  w = c * NS + t
  ones = jnp.full((L,), 1.0, jnp.float32)
  zeros = jnp.zeros((L,), jnp.float32)

  def zero_hist(i, carry):
    histv[pl.ds(i * L, L)] = zeros
    return carry
  lax.fori_loop(0, NPAD // L, zero_hist, 0)

  pltpu.sync_copy(dst_hbm.at[pl.ds(w * EPB, EPB)], dstv)

  def hist_step(i, carry):
    idx = dstv[pl.ds(i * L, L)]
    plsc.addupdate_scatter(histv, [idx], ones)
    return carry
  lax.fori_loop(0, EPB // L, hist_step, 0)

  pltpu.sync_copy(histv, shared.at[t])
  plsc.subcore_barrier()

  cb = t * ROWS_PER_TEC
  for k in range(NS):
    pltpu.sync_copy(shared.at[k, pl.ds(cb, ROWS_PER_TEC)], tmpv.at[k])

  def col_sum(i, carry):
    acc = tmpv[0, pl.ds(i * L, L)]
    for k in range(1, NS):
      acc = acc + tmpv[k, pl.ds(i * L, L)]
    colv[pl.ds(i * L, L)] = acc
    return carry
  lax.fori_loop(0, ROWS_PER_TEC // L, col_sum, 0)

  pltpu.sync_copy(colv, out_hbm.at[pl.ds(c * NPAD + cb, ROWS_PER_TEC)])


@functools.lru_cache(maxsize=None)
def _deg_kernel():
  return pl.kernel(
    _deg_body,
    out_type=jax.ShapeDtypeStruct((NC * NPAD,), jnp.float32),
    mesh=_mesh(),
    scratch_types=[
        pltpu.VMEM((EPB,), jnp.int32),
        pltpu.VMEM((NPAD,), jnp.float32),
        pltpu.VMEM((NS, ROWS_PER_TEC), jnp.float32),
        pltpu.VMEM((ROWS_PER_TEC,), jnp.float32),
        pltpu.VMEM_SHARED((NS, NPAD), jnp.float32),
    ],
    compiler_params=pltpu.CompilerParams(needs_layout_passes=False),
  )


CB = 64
NCH = EPAD // (NS * CB)


@functools.lru_cache(maxsize=None)
def _make_prop(S):
  s_half = S // NC
  epb = NCH * CB
  nhalf = NCH // 2

  def body(h_hbm, src_hbm, dst2_hbm, zeros_hbm, out_hbm,
           srcf, dstb, bf_v, fbuf, acc_sh, sem_ga, sem_gb, sem_sa, sem_sb):
    c = lax.axis_index("c")
    t = lax.axis_index("s")
    my_rows = t * ROWS_PER_TEC
    gsems = (sem_ga, sem_gb)
    ssems = (sem_sa, sem_sb)

    pltpu.sync_copy(src_hbm.at[pl.ds(t * epb, epb)], srcf)
    pltpu.sync_copy(dst2_hbm.at[pl.ds(t * epb, epb)], dstb)

    def adjust(delta):
      dv = jnp.full((L,), delta, jnp.int32)
      def go(i, carry):
        srcf[pl.ds(i * L, L)] = srcf[pl.ds(i * L, L)] + dv
        return carry
      lax.fori_loop(0, epb // L, go, 0)

    def gidx(j):
      return srcf.at[pl.ds(j * CB, CB)]

    def bfb(p):
      return bf_v.at[pl.ds(p * CB, CB)]

    def fb(p):
      return fbuf.at[pl.ds(p * CB, CB)]

    def issue_g(ch, p):
      pltpu.async_copy(h_hbm.at[gidx(ch)], bfb(p), gsems[p])

    def wait_g(p):
      pltpu.make_async_copy(h_hbm.at[gidx(0)], bfb(p), gsems[p]).wait()

    def issue_s(ch, p):
      pltpu.async_copy(fb(p), acc_sh.at[dstb.at[pl.ds(ch * CB, CB)]],
                       ssems[p], add=True)

    def wait_s(p):
      pltpu.make_async_copy(fb(p), acc_sh.at[dstb.at[pl.ds(0, CB)]],
                            ssems[p]).wait()

    def convert(p):
      mask = jnp.full((L,), -65536, jnp.int32)
      sh = jnp.full((L,), 16, jnp.int32)
      def row(r4, carry):
        for u in range(4):
          r = 4 * r4 + u
          for g in range(4):
            y = bf_v[p * CB + r, pl.ds(g * L, L)]
            fbuf[p * CB + r, pl.ds(g * 2 * L, L)] = plsc.bitcast(
                jnp.left_shift(y, sh), jnp.float32)
            fbuf[p * CB + r, pl.ds(g * 2 * L + L, L)] = plsc.bitcast(
                y & mask, jnp.float32)
        return carry
      lax.fori_loop(0, CB // 4, row, 0)

    adjust(c * (s_half * N))
    for si in range(s_half):
      if si:
        adjust(N)
      issue_g(0, 0)
      pltpu.sync_copy(zeros_hbm, acc_sh.at[pl.ds(my_rows, ROWS_PER_TEC)])
      plsc.subcore_barrier()

      def pipe(j, carry):
        j0 = 2 * j
        issue_g(j0 + 1, 1)
        wait_g(0)
        @pl.when(j > 0)
        def _():
          wait_s(0)
        convert(0)
        issue_s(j0, 0)
        @pl.when(j < nhalf - 1)
        def _():
          issue_g(j0 + 2, 0)
        wait_g(1)
        @pl.when(j > 0)
        def _():
          wait_s(1)
        convert(1)
        issue_s(j0 + 1, 1)
        return carry
      lax.fori_loop(0, nhalf, pipe, 0)

      wait_s(0)
      wait_s(1)
      plsc.subcore_barrier()
      ob = (c * s_half + si) * NPAD + my_rows
      pltpu.sync_copy(acc_sh.at[pl.ds(my_rows, ROWS_PER_TEC)],
                      out_hbm.at[pl.ds(ob, ROWS_PER_TEC)])

  return pl.kernel(
      body,
      out_type=jax.ShapeDtypeStruct((S * NPAD, 128), jnp.float32),
      mesh=_mesh(),
      scratch_types=[
          pltpu.VMEM((epb,), jnp.int32),
          pltpu.VMEM((epb,), jnp.int32),
          pltpu.VMEM((2 * CB, 64), jnp.int32),
          pltpu.VMEM((2 * CB, 128), jnp.float32),
          pltpu.VMEM_SHARED((NPAD, 128), jnp.float32),
          pltpu.SemaphoreType.DMA,
          pltpu.SemaphoreType.DMA,
          pltpu.SemaphoreType.DMA,
          pltpu.SemaphoreType.DMA,
      ],
      compiler_params=pltpu.CompilerParams(needs_layout_passes=False,
                                           use_tc_tiling_on_sc=False),
  )


_BN = 1000
_GRID = N // _BN


def _pack128(h128):
  words = []
  for i in range(4):
    lo = h128[:, 32 * i:32 * i + 16]
    hi = h128[:, 32 * i + 16:32 * i + 32]
    lo16 = lax.bitcast_convert_type(lo.astype(jnp.bfloat16),
                                    jnp.uint16).astype(jnp.int32)
    hi16 = lax.bitcast_convert_type(hi.astype(jnp.bfloat16),
                                    jnp.uint16).astype(jnp.int32)
    words.append((hi16 << 16) | lo16)
  return jnp.concatenate(words, axis=1)


def _mm1_body(x_ref, w_ref, d0_ref, d1_ref, hsl_ref, hslb_ref, dinv_ref):
  dinv = lax.rsqrt(1.0 + d0_ref[...] + d1_ref[...])
  dinv_ref[...] = dinv
  h = jnp.dot(x_ref[...], w_ref[...], preferred_element_type=jnp.float32)
  h = h * dinv
  for s in range(8):
    hsl_ref[s] = h[:, 128 * s:128 * (s + 1)]
    hslb_ref[s] = _pack128(h[:, 128 * s:128 * (s + 1)])


def _mm1_call(x, wcat, d0, d1):
  return pl.pallas_call(
      _mm1_body,
      grid=(_GRID,),
      in_specs=[
          pl.BlockSpec((_BN, 256), lambda i: (i, 0)),
          pl.BlockSpec((256, 1024), lambda i: (0, 0)),
          pl.BlockSpec((_BN, 1), lambda i: (i, 0)),
          pl.BlockSpec((_BN, 1), lambda i: (i, 0)),
      ],
      out_specs=[
          pl.BlockSpec((8, _BN, 128), lambda i: (0, i, 0)),
          pl.BlockSpec((8, _BN, 64), lambda i: (0, i, 0)),
          pl.BlockSpec((_BN, 1), lambda i: (i, 0)),
      ],
      out_shape=[
          jax.ShapeDtypeStruct((8, N, 128), jnp.float32),
          jax.ShapeDtypeStruct((8, N, 64), jnp.int32),
          jax.ShapeDtypeStruct((N, 1), jnp.float32),
      ],
  )(x, wcat, d0, d1)


def _ef_body(acc_ref, hsl_ref, dinv_ref, b_ref, w_ref,
             x1_ref, h1_ref, hsl2_ref, hslb2_ref):
  dinv = dinv_ref[...]
  zs = []
  for s in range(8):
    z = dinv * (acc_ref[s] + hsl_ref[s]) + b_ref[s]
    zs.append(jnp.maximum(z, 0.0))
  x1 = jnp.concatenate(zs[:4], axis=1)
  h1 = jnp.concatenate(zs[4:], axis=1)
  x1_ref[...] = x1
  h1_ref[...] = h1
  hh = jnp.dot(h1, w_ref[...], preferred_element_type=jnp.float32) * dinv
  for s in range(4):
    hsl2_ref[s] = hh[:, 128 * s:128 * (s + 1)]
    hslb2_ref[s] = _pack128(hh[:, 128 * s:128 * (s + 1)])


def _ef_call(acc1, hsl1, dinv, bcat, w2m):
  return pl.pallas_call(
      _ef_body,
      grid=(_GRID,),
      in_specs=[
          pl.BlockSpec((8, _BN, 128), lambda i: (0, i, 0)),
          pl.BlockSpec((8, _BN, 128), lambda i: (0, i, 0)),
          pl.BlockSpec((_BN, 1), lambda i: (i, 0)),
          pl.BlockSpec((8, 1, 128), lambda i: (0, 0, 0)),
          pl.BlockSpec((512, 512), lambda i: (0, 0)),
      ],
      out_specs=[
          pl.BlockSpec((_BN, 512), lambda i: (i, 0)),
          pl.BlockSpec((_BN, 512), lambda i: (i, 0)),
          pl.BlockSpec((4, _BN, 128), lambda i: (0, i, 0)),
          pl.BlockSpec((4, _BN, 64), lambda i: (0, i, 0)),
      ],
      out_shape=[
          jax.ShapeDtypeStruct((N, 512), jnp.float32),
          jax.ShapeDtypeStruct((N, 512), jnp.float32),
          jax.ShapeDtypeStruct((4, N, 128), jnp.float32),
          jax.ShapeDtypeStruct((4, N, 64), jnp.int32),
      ],
  )(acc1, hsl1, dinv, bcat, w2m)


def _gh_body(acc_ref, hsl_ref, dinv_ref, b_ref, x1_ref, wa_ref, wb_ref,
             h2_ref, hsl3_ref, hslb3_ref):
  dinv = dinv_ref[...]
  zs = []
  for s in range(4):
    z = dinv * (acc_ref[s] + hsl_ref[s]) + b_ref[s]
    zs.append(jnp.maximum(z, 0.0))
  h2 = jnp.concatenate(zs, axis=1)
  h2_ref[...] = h2
  y = (jnp.dot(x1_ref[...], wa_ref[...], preferred_element_type=jnp.float32)
       + jnp.dot(h2, wb_ref[...], preferred_element_type=jnp.float32)) * dinv
  for s in range(2):
    hsl3_ref[s] = y[:, 128 * s:128 * (s + 1)]
    hslb3_ref[s] = _pack128(y[:, 128 * s:128 * (s + 1)])


def _gh_call(acc2, hsl2, dinv, b2, x1, wa, wb):
  return pl.pallas_call(
      _gh_body,
      grid=(_GRID,),
      in_specs=[
          pl.BlockSpec((4, _BN, 128), lambda i: (0, i, 0)),
          pl.BlockSpec((4, _BN, 128), lambda i: (0, i, 0)),
          pl.BlockSpec((_BN, 1), lambda i: (i, 0)),
          pl.BlockSpec((4, 1, 128), lambda i: (0, 0, 0)),
          pl.BlockSpec((_BN, 512), lambda i: (i, 0)),
          pl.BlockSpec((512, 256), lambda i: (0, 0)),
          pl.BlockSpec((512, 256), lambda i: (0, 0)),
      ],
      out_specs=[
          pl.BlockSpec((_BN, 512), lambda i: (i, 0)),
          pl.BlockSpec((2, _BN, 128), lambda i: (0, i, 0)),
          pl.BlockSpec((2, _BN, 64), lambda i: (0, i, 0)),
      ],
      out_shape=[
          jax.ShapeDtypeStruct((N, 512), jnp.float32),
          jax.ShapeDtypeStruct((2, N, 128), jnp.float32),
          jax.ShapeDtypeStruct((2, N, 64), jnp.int32),
      ],
  )(acc2, hsl2, dinv, b2, x1, wa, wb)


def _ep3_body(acc_ref, hsl_ref, dinv_ref, b_ref, out_ref):
  dinv = dinv_ref[...]
  for s in range(2):
    out_ref[:, 128 * s:128 * (s + 1)] = (
        dinv * (acc_ref[s] + hsl_ref[s]) + b_ref[s])


def _ep3_call(acc3, hsl3, dinv, bo):
  return pl.pallas_call(
      _ep3_body,
      grid=(_GRID,),
      in_specs=[
          pl.BlockSpec((2, _BN, 128), lambda i: (0, i, 0)),
          pl.BlockSpec((2, _BN, 128), lambda i: (0, i, 0)),
          pl.BlockSpec((_BN, 1), lambda i: (i, 0)),
          pl.BlockSpec((2, 1, 128), lambda i: (0, 0, 0)),
      ],
      out_specs=pl.BlockSpec((_BN, 256), lambda i: (i, 0)),
      out_shape=jax.ShapeDtypeStruct((N, 256), jnp.float32),
  )(acc3, hsl3, dinv, bo)


@jax.jit
def _run(x, edge_index, W1h, b1h, W2h, b2h, W2m, b2m, Wout, bout):
  src = edge_index[0]
  dst = edge_index[1]
  srcp = jnp.concatenate([src, jnp.zeros((EPAD - E,), jnp.int32)])
  dstp = jnp.concatenate([dst, jnp.full((EPAD - E,), NPAD - 1, jnp.int32)])
  dst2 = dstp
  zeros128 = jnp.zeros((ROWS_PER_TEC, 128), jnp.float32)

  degp = _deg_kernel()(dstp)
  d0 = degp[:N].reshape(N, 1)
  d1 = degp[NPAD:NPAD + N].reshape(N, 1)

  wcat = jnp.concatenate([W1h, W2h], axis=1)
  bcat = jnp.concatenate([b1h, b2h]).reshape(8, 1, 128)

  hsl1, hslb1, dinv = _mm1_call(x, wcat, d0, d1)
  acc1 = _make_prop(8)(hslb1.reshape(8 * N, 64), srcp, dst2, zeros128)
  x1, h1, hsl2, hslb2 = _ef_call(acc1.reshape(8, NPAD, 128), hsl1, dinv,
                                 bcat, W2m)

  acc2 = _make_prop(4)(hslb2.reshape(4 * N, 64), srcp, dst2, zeros128)
  h2, hsl3, hslb3 = _gh_call(acc2.reshape(4, NPAD, 128), hsl2, dinv,
                             b2m.reshape(4, 1, 128), x1, Wout[:512],
                             Wout[512:])

  acc3 = _make_prop(2)(hslb3.reshape(2 * N, 64), srcp, dst2, zeros128)
  out = _ep3_call(acc3.reshape(2, NPAD, 128), hsl3, dinv,
                  bout.reshape(2, 1, 128))
  return out, x1, h1, h2


def kernel(x, edge_index, percent, ricci_curvature,
           W1h, b1h, W2h, b2h, W2m, b2m, Wout, bout):
  del percent, ricci_curvature
  return _run(x, edge_index, W1h, b1h, W2h, b2h, W2m, b2m, Wout, bout)


# speedup vs baseline: 1.0030x; 1.0003x over previous
import functools

import jax
import jax.numpy as jnp
from jax import lax
from jax.experimental import pallas as pl
from jax.experimental.pallas import tpu as pltpu
from jax.experimental.pallas import tpu_sc as plsc

N = 10000
E = 160000
NC = 2
NS = 16
L = 16
NPAD = 10240
EPAD = 163840
CH = 128
ROWS_PER_TEC = NPAD // NS
CHUNKS_TOTAL = EPAD // CH
CHUNKS_PER_TEC = CHUNKS_TOTAL // NS
EPB = EPAD // (NC * NS)

@functools.lru_cache(maxsize=None)
def _mesh():
  return plsc.VectorSubcoreMesh(
      core_axis_name="c", subcore_axis_name="s", num_cores=NC, num_subcores=NS)


def _deg_body(dst_hbm, out_hbm, dstv, histv, tmpv, colv, shared):
  c = lax.axis_index("c")
  t = lax.axis_index("s")
  w = c * NS + t
  ones = jnp.full((L,), 1.0, jnp.float32)
  zeros = jnp.zeros((L,), jnp.float32)

  def zero_hist(i, carry):
    histv[pl.ds(i * L, L)] = zeros
    return carry
  lax.fori_loop(0, NPAD // L, zero_hist, 0)

  pltpu.sync_copy(dst_hbm.at[pl.ds(w * EPB, EPB)], dstv)

  def hist_step(i, carry):
    idx = dstv[pl.ds(i * L, L)]
    plsc.addupdate_scatter(histv, [idx], ones)
    return carry
  lax.fori_loop(0, EPB // L, hist_step, 0)

  pltpu.sync_copy(histv, shared.at[t])
  plsc.subcore_barrier()

  cb = t * ROWS_PER_TEC
  for k in range(NS):
    pltpu.sync_copy(shared.at[k, pl.ds(cb, ROWS_PER_TEC)], tmpv.at[k])

  def col_sum(i, carry):
    acc = tmpv[0, pl.ds(i * L, L)]
    for k in range(1, NS):
      acc = acc + tmpv[k, pl.ds(i * L, L)]
    colv[pl.ds(i * L, L)] = acc
    return carry
  lax.fori_loop(0, ROWS_PER_TEC // L, col_sum, 0)

  pltpu.sync_copy(colv, out_hbm.at[pl.ds(c * NPAD + cb, ROWS_PER_TEC)])


@functools.lru_cache(maxsize=None)
def _deg_kernel():
  return pl.kernel(
    _deg_body,
    out_type=jax.ShapeDtypeStruct((NC * NPAD,), jnp.float32),
    mesh=_mesh(),
    scratch_types=[
        pltpu.VMEM((EPB,), jnp.int32),
        pltpu.VMEM((NPAD,), jnp.float32),
        pltpu.VMEM((NS, ROWS_PER_TEC), jnp.float32),
        pltpu.VMEM((ROWS_PER_TEC,), jnp.float32),
        pltpu.VMEM_SHARED((NS, NPAD), jnp.float32),
    ],
    compiler_params=pltpu.CompilerParams(needs_layout_passes=False),
  )


CB = 64
NCH = EPAD // (NS * CB)


@functools.lru_cache(maxsize=None)
def _make_prop(S):
  s_half = S // NC
  epb = NCH * CB
  nhalf = NCH // 2

  def body(h_hbm, src_hbm, dst2_hbm, zeros_hbm, out_hbm,
           srcf, dstb, bf_v, fbuf, acc_sh, sem_ga, sem_gb, sem_sa, sem_sb):
    c = lax.axis_index("c")
    t = lax.axis_index("s")
    my_rows = t * ROWS_PER_TEC
    gsems = (sem_ga, sem_gb)
    ssems = (sem_sa, sem_sb)

    pltpu.sync_copy(src_hbm.at[pl.ds(t * epb, epb)], srcf)
    pltpu.sync_copy(dst2_hbm.at[pl.ds(t * epb, epb)], dstb)

    def adjust(delta):
      dv = jnp.full((L,), delta, jnp.int32)
      def go(i, carry):
        srcf[pl.ds(i * L, L)] = srcf[pl.ds(i * L, L)] + dv
        return carry
      lax.fori_loop(0, epb // L, go, 0)

    def gidx(j):
      return srcf.at[pl.ds(j * CB, CB)]

    def bfb(p):
      return bf_v.at[pl.ds(p * CB, CB)]

    def fb(p):
      return fbuf.at[pl.ds(p * CB, CB)]

    def issue_g(ch, p):
      pltpu.async_copy(h_hbm.at[gidx(ch)], bfb(p), gsems[p])

    def wait_g(p):
      pltpu.make_async_copy(h_hbm.at[gidx(0)], bfb(p), gsems[p]).wait()

    def issue_s(ch, p):
      pltpu.async_copy(fb(p), acc_sh.at[dstb.at[pl.ds(ch * CB, CB)]],
                       ssems[p], add=True)

    def wait_s(p):
      pltpu.make_async_copy(fb(p), acc_sh.at[dstb.at[pl.ds(0, CB)]],
                            ssems[p]).wait()

    def convert(p):
      mask = jnp.full((L,), -65536, jnp.int32)
      sh = jnp.full((L,), 16, jnp.int32)
      def row(r, carry):
        for g in range(4):
          y = bf_v[p * CB + r, pl.ds(g * L, L)]
          fbuf[p * CB + r, pl.ds(g * 2 * L, L)] = plsc.bitcast(
              jnp.left_shift(y, sh), jnp.float32)
          fbuf[p * CB + r, pl.ds(g * 2 * L + L, L)] = plsc.bitcast(
              y & mask, jnp.float32)
        return carry
      lax.fori_loop(0, CB, row, 0)

    adjust(c * (s_half * N))
    for si in range(s_half):
      if si:
        adjust(N)
      issue_g(0, 0)
      pltpu.sync_copy(zeros_hbm, acc_sh.at[pl.ds(my_rows, ROWS_PER_TEC)])
      plsc.subcore_barrier()

      def pipe(j, carry):
        j0 = 2 * j
        issue_g(j0 + 1, 1)
        wait_g(0)
        @pl.when(j > 0)
        def _():
          wait_s(0)
        convert(0)
        issue_s(j0, 0)
        @pl.when(j < nhalf - 1)
        def _():
          issue_g(j0 + 2, 0)
        wait_g(1)
        @pl.when(j > 0)
        def _():
          wait_s(1)
        convert(1)
        issue_s(j0 + 1, 1)
        return carry
      lax.fori_loop(0, nhalf, pipe, 0)

      wait_s(0)
      wait_s(1)
      plsc.subcore_barrier()
      ob = (c * s_half + si) * NPAD + my_rows
      pltpu.sync_copy(acc_sh.at[pl.ds(my_rows, ROWS_PER_TEC)],
                      out_hbm.at[pl.ds(ob, ROWS_PER_TEC)])

  return pl.kernel(
      body,
      out_type=jax.ShapeDtypeStruct((S * NPAD, 128), jnp.float32),
      mesh=_mesh(),
      scratch_types=[
          pltpu.VMEM((epb,), jnp.int32),
          pltpu.VMEM((epb,), jnp.int32),
          pltpu.VMEM((2 * CB, 64), jnp.int32),
          pltpu.VMEM((2 * CB, 128), jnp.float32),
          pltpu.VMEM_SHARED((NPAD, 128), jnp.float32),
          pltpu.SemaphoreType.DMA,
          pltpu.SemaphoreType.DMA,
          pltpu.SemaphoreType.DMA,
          pltpu.SemaphoreType.DMA,
      ],
      compiler_params=pltpu.CompilerParams(needs_layout_passes=False,
                                           use_tc_tiling_on_sc=False),
  )


_BN = 1000
_GRID = N // _BN


def _pack128(h128):
  words = []
  for i in range(4):
    lo = h128[:, 32 * i:32 * i + 16]
    hi = h128[:, 32 * i + 16:32 * i + 32]
    lo16 = lax.bitcast_convert_type(lo.astype(jnp.bfloat16),
                                    jnp.uint16).astype(jnp.int32)
    hi16 = lax.bitcast_convert_type(hi.astype(jnp.bfloat16),
                                    jnp.uint16).astype(jnp.int32)
    words.append((hi16 << 16) | lo16)
  return jnp.concatenate(words, axis=1)


def _mm1_body(x_ref, w_ref, d0_ref, d1_ref, hsl_ref, hslb_ref, dinv_ref):
  dinv = lax.rsqrt(1.0 + d0_ref[...] + d1_ref[...])
  dinv_ref[...] = dinv
  h = jnp.dot(x_ref[...], w_ref[...], preferred_element_type=jnp.float32)
  h = h * dinv
  for s in range(8):
    hsl_ref[s] = h[:, 128 * s:128 * (s + 1)]
    hslb_ref[s] = _pack128(h[:, 128 * s:128 * (s + 1)])


def _mm1_call(x, wcat, d0, d1):
  return pl.pallas_call(
      _mm1_body,
      grid=(_GRID,),
      in_specs=[
          pl.BlockSpec((_BN, 256), lambda i: (i, 0)),
          pl.BlockSpec((256, 1024), lambda i: (0, 0)),
          pl.BlockSpec((_BN, 1), lambda i: (i, 0)),
          pl.BlockSpec((_BN, 1), lambda i: (i, 0)),
      ],
      out_specs=[
          pl.BlockSpec((8, _BN, 128), lambda i: (0, i, 0)),
          pl.BlockSpec((8, _BN, 64), lambda i: (0, i, 0)),
          pl.BlockSpec((_BN, 1), lambda i: (i, 0)),
      ],
      out_shape=[
          jax.ShapeDtypeStruct((8, N, 128), jnp.float32),
          jax.ShapeDtypeStruct((8, N, 64), jnp.int32),
          jax.ShapeDtypeStruct((N, 1), jnp.float32),
      ],
  )(x, wcat, d0, d1)


def _ef_body(acc_ref, hsl_ref, dinv_ref, b_ref, w_ref,
             x1_ref, h1_ref, hsl2_ref, hslb2_ref):
  dinv = dinv_ref[...]
  zs = []
  for s in range(8):
    z = dinv * (acc_ref[s] + hsl_ref[s]) + b_ref[s]
    zs.append(jnp.maximum(z, 0.0))
  x1 = jnp.concatenate(zs[:4], axis=1)
  h1 = jnp.concatenate(zs[4:], axis=1)
  x1_ref[...] = x1
  h1_ref[...] = h1
  hh = jnp.dot(h1, w_ref[...], preferred_element_type=jnp.float32) * dinv
  for s in range(4):
    hsl2_ref[s] = hh[:, 128 * s:128 * (s + 1)]
    hslb2_ref[s] = _pack128(hh[:, 128 * s:128 * (s + 1)])


def _ef_call(acc1, hsl1, dinv, bcat, w2m):
  return pl.pallas_call(
      _ef_body,
      grid=(_GRID,),
      in_specs=[
          pl.BlockSpec((8, _BN, 128), lambda i: (0, i, 0)),
          pl.BlockSpec((8, _BN, 128), lambda i: (0, i, 0)),
          pl.BlockSpec((_BN, 1), lambda i: (i, 0)),
          pl.BlockSpec((8, 1, 128), lambda i: (0, 0, 0)),
          pl.BlockSpec((512, 512), lambda i: (0, 0)),
      ],
      out_specs=[
          pl.BlockSpec((_BN, 512), lambda i: (i, 0)),
          pl.BlockSpec((_BN, 512), lambda i: (i, 0)),
          pl.BlockSpec((4, _BN, 128), lambda i: (0, i, 0)),
          pl.BlockSpec((4, _BN, 64), lambda i: (0, i, 0)),
      ],
      out_shape=[
          jax.ShapeDtypeStruct((N, 512), jnp.float32),
          jax.ShapeDtypeStruct((N, 512), jnp.float32),
          jax.ShapeDtypeStruct((4, N, 128), jnp.float32),
          jax.ShapeDtypeStruct((4, N, 64), jnp.int32),
      ],
  )(acc1, hsl1, dinv, bcat, w2m)


def _gh_body(acc_ref, hsl_ref, dinv_ref, b_ref, x1_ref, wa_ref, wb_ref,
             h2_ref, hsl3_ref, hslb3_ref):
  dinv = dinv_ref[...]
  zs = []
  for s in range(4):
    z = dinv * (acc_ref[s] + hsl_ref[s]) + b_ref[s]
    zs.append(jnp.maximum(z, 0.0))
  h2 = jnp.concatenate(zs, axis=1)
  h2_ref[...] = h2
  y = (jnp.dot(x1_ref[...], wa_ref[...], preferred_element_type=jnp.float32)
       + jnp.dot(h2, wb_ref[...], preferred_element_type=jnp.float32)) * dinv
  for s in range(2):
    hsl3_ref[s] = y[:, 128 * s:128 * (s + 1)]
    hslb3_ref[s] = _pack128(y[:, 128 * s:128 * (s + 1)])


def _gh_call(acc2, hsl2, dinv, b2, x1, wa, wb):
  return pl.pallas_call(
      _gh_body,
      grid=(_GRID,),
      in_specs=[
          pl.BlockSpec((4, _BN, 128), lambda i: (0, i, 0)),
          pl.BlockSpec((4, _BN, 128), lambda i: (0, i, 0)),
          pl.BlockSpec((_BN, 1), lambda i: (i, 0)),
          pl.BlockSpec((4, 1, 128), lambda i: (0, 0, 0)),
          pl.BlockSpec((_BN, 512), lambda i: (i, 0)),
          pl.BlockSpec((512, 256), lambda i: (0, 0)),
          pl.BlockSpec((512, 256), lambda i: (0, 0)),
      ],
      out_specs=[
          pl.BlockSpec((_BN, 512), lambda i: (i, 0)),
          pl.BlockSpec((2, _BN, 128), lambda i: (0, i, 0)),
          pl.BlockSpec((2, _BN, 64), lambda i: (0, i, 0)),
      ],
      out_shape=[
          jax.ShapeDtypeStruct((N, 512), jnp.float32),
          jax.ShapeDtypeStruct((2, N, 128), jnp.float32),
          jax.ShapeDtypeStruct((2, N, 64), jnp.int32),
      ],
  )(acc2, hsl2, dinv, b2, x1, wa, wb)


def _ep3_body(acc_ref, hsl_ref, dinv_ref, b_ref, out_ref):
  dinv = dinv_ref[...]
  for s in range(2):
    out_ref[:, 128 * s:128 * (s + 1)] = (
        dinv * (acc_ref[s] + hsl_ref[s]) + b_ref[s])


def _ep3_call(acc3, hsl3, dinv, bo):
  return pl.pallas_call(
      _ep3_body,
      grid=(_GRID,),
      in_specs=[
          pl.BlockSpec((2, _BN, 128), lambda i: (0, i, 0)),
          pl.BlockSpec((2, _BN, 128), lambda i: (0, i, 0)),
          pl.BlockSpec((_BN, 1), lambda i: (i, 0)),
          pl.BlockSpec((2, 1, 128), lambda i: (0, 0, 0)),
      ],
      out_specs=pl.BlockSpec((_BN, 256), lambda i: (i, 0)),
      out_shape=jax.ShapeDtypeStruct((N, 256), jnp.float32),
  )(acc3, hsl3, dinv, bo)


@jax.jit
def _run(x, edge_index, W1h, b1h, W2h, b2h, W2m, b2m, Wout, bout):
  src = edge_index[0]
  dst = edge_index[1]
  srcp = jnp.concatenate([src, jnp.zeros((EPAD - E,), jnp.int32)])
  dstp = jnp.concatenate([dst, jnp.full((EPAD - E,), NPAD - 1, jnp.int32)])
  dst2 = dstp
  zeros128 = jnp.zeros((ROWS_PER_TEC, 128), jnp.float32)

  degp = _deg_kernel()(dstp)
  d0 = degp[:N].reshape(N, 1)
  d1 = degp[NPAD:NPAD + N].reshape(N, 1)

  wcat = jnp.concatenate([W1h, W2h], axis=1)
  bcat = jnp.concatenate([b1h, b2h]).reshape(8, 1, 128)

  hsl1, hslb1, dinv = _mm1_call(x, wcat, d0, d1)
  acc1 = _make_prop(8)(hslb1.reshape(8 * N, 64), srcp, dst2, zeros128)
  x1, h1, hsl2, hslb2 = _ef_call(acc1.reshape(8, NPAD, 128), hsl1, dinv,
                                 bcat, W2m)

  acc2 = _make_prop(4)(hslb2.reshape(4 * N, 64), srcp, dst2, zeros128)
  h2, hsl3, hslb3 = _gh_call(acc2.reshape(4, NPAD, 128), hsl2, dinv,
                             b2m.reshape(4, 1, 128), x1, Wout[:512],
                             Wout[512:])

  acc3 = _make_prop(2)(hslb3.reshape(2 * N, 64), srcp, dst2, zeros128)
  out = _ep3_call(acc3.reshape(2, NPAD, 128), hsl3, dinv,
                  bout.reshape(2, 1, 128))
  return out, x1, h1, h2


def kernel(x, edge_index, percent, ricci_curvature,
           W1h, b1h, W2h, b2h, W2m, b2m, Wout, bout):
  del percent, ricci_curvature
  return _run(x, edge_index, W1h, b1h, W2h, b2h, W2m, b2m, Wout, bout)


# speedup vs baseline: 1.0193x; 1.0162x over previous
import functools

import jax
import jax.numpy as jnp
from jax import lax
from jax.experimental import pallas as pl
from jax.experimental.pallas import tpu as pltpu
from jax.experimental.pallas import tpu_sc as plsc

N = 10000
E = 160000
NC = 2
NS = 16
L = 16
NPAD = 10240
EPAD = 163840
CH = 128
ROWS_PER_TEC = NPAD // NS
CHUNKS_TOTAL = EPAD // CH
CHUNKS_PER_TEC = CHUNKS_TOTAL // NS
EPB = EPAD // (NC * NS)

@functools.lru_cache(maxsize=None)
def _mesh():
  return plsc.VectorSubcoreMesh(
      core_axis_name="c", subcore_axis_name="s", num_cores=NC, num_subcores=NS)


def _deg_body(dst_hbm, out_hbm, dstv, histv, tmpv, colv, shared):
  c = lax.axis_index("c")
  t = lax.axis_index("s")
  w = c * NS + t
  ones = jnp.full((L,), 1.0, jnp.float32)
  zeros = jnp.zeros((L,), jnp.float32)

  def zero_hist(i, carry):
    histv[pl.ds(i * L, L)] = zeros
    return carry
  lax.fori_loop(0, NPAD // L, zero_hist, 0)

  pltpu.sync_copy(dst_hbm.at[pl.ds(w * EPB, EPB)], dstv)

  def hist_step(i, carry):
    idx = dstv[pl.ds(i * L, L)]
    plsc.addupdate_scatter(histv, [idx], ones)
    return carry
  lax.fori_loop(0, EPB // L, hist_step, 0)

  pltpu.sync_copy(histv, shared.at[t])
  plsc.subcore_barrier()

  cb = t * ROWS_PER_TEC
  for k in range(NS):
    pltpu.sync_copy(shared.at[k, pl.ds(cb, ROWS_PER_TEC)], tmpv.at[k])

  def col_sum(i, carry):
    acc = tmpv[0, pl.ds(i * L, L)]
    for k in range(1, NS):
      acc = acc + tmpv[k, pl.ds(i * L, L)]
    colv[pl.ds(i * L, L)] = acc
    return carry
  lax.fori_loop(0, ROWS_PER_TEC // L, col_sum, 0)

  pltpu.sync_copy(colv, out_hbm.at[pl.ds(c * NPAD + cb, ROWS_PER_TEC)])


@functools.lru_cache(maxsize=None)
def _deg_kernel():
  return pl.kernel(
    _deg_body,
    out_type=jax.ShapeDtypeStruct((NC * NPAD,), jnp.float32),
    mesh=_mesh(),
    scratch_types=[
        pltpu.VMEM((EPB,), jnp.int32),
        pltpu.VMEM((NPAD,), jnp.float32),
        pltpu.VMEM((NS, ROWS_PER_TEC), jnp.float32),
        pltpu.VMEM((ROWS_PER_TEC,), jnp.float32),
        pltpu.VMEM_SHARED((NS, NPAD), jnp.float32),
    ],
    compiler_params=pltpu.CompilerParams(needs_layout_passes=False),
  )


CB = 32
NCH = EPAD // (NS * CB)
GBUF = 4


@functools.lru_cache(maxsize=None)
def _make_prop(S):
  s_half = S // NC
  epb = NCH * CB
  nhalf = NCH // 2

  def body(h_hbm, src_hbm, dst2_hbm, zeros_hbm, out_hbm,
           srcf, dstb, bf_v, fbuf, acc_sh,
           sem_ga, sem_gb, sem_gc, sem_gd, sem_sa, sem_sb):
    c = lax.axis_index("c")
    t = lax.axis_index("s")
    my_rows = t * ROWS_PER_TEC
    gsems = (sem_ga, sem_gb, sem_gc, sem_gd)
    ssems = (sem_sa, sem_sb)

    pltpu.sync_copy(src_hbm.at[pl.ds(t * epb, epb)], srcf)
    pltpu.sync_copy(dst2_hbm.at[pl.ds(t * epb, epb)], dstb)

    def adjust(delta):
      dv = jnp.full((L,), delta, jnp.int32)
      def go(i, carry):
        srcf[pl.ds(i * L, L)] = srcf[pl.ds(i * L, L)] + dv
        return carry
      lax.fori_loop(0, epb // L, go, 0)

    def gidx(j):
      return srcf.at[pl.ds(j * CB, CB)]

    def bfb(p):
      return bf_v.at[pl.ds(p * CB, CB)]

    def fb(p):
      return fbuf.at[pl.ds(p * CB, CB)]

    def issue_g(ch, p):
      pltpu.async_copy(h_hbm.at[gidx(ch)], bfb(p), gsems[p])

    def wait_g(p):
      pltpu.make_async_copy(h_hbm.at[gidx(0)], bfb(p), gsems[p]).wait()

    def issue_s(ch, p):
      pltpu.async_copy(fb(p), acc_sh.at[dstb.at[pl.ds(ch * CB, CB)]],
                       ssems[p], add=True)

    def wait_s(p):
      pltpu.make_async_copy(fb(p), acc_sh.at[dstb.at[pl.ds(0, CB)]],
                            ssems[p]).wait()

    def convert(pb, pf):
      mask = jnp.full((L,), -65536, jnp.int32)
      sh = jnp.full((L,), 16, jnp.int32)
      def row(r, carry):
        for g in range(4):
          y = bf_v[pb * CB + r, pl.ds(g * L, L)]
          fbuf[pf * CB + r, pl.ds(g * 2 * L, L)] = plsc.bitcast(
              jnp.left_shift(y, sh), jnp.float32)
          fbuf[pf * CB + r, pl.ds(g * 2 * L + L, L)] = plsc.bitcast(
              y & mask, jnp.float32)
        return carry
      lax.fori_loop(0, CB, row, 0)

    adjust(c * (s_half * N))
    for si in range(s_half):
      if si:
        adjust(N)
      for p in range(GBUF - 1):
        issue_g(p, p)
      pltpu.sync_copy(zeros_hbm, acc_sh.at[pl.ds(my_rows, ROWS_PER_TEC)])
      plsc.subcore_barrier()

      def pipe(j, carry):
        for p in range(GBUF):
          ch = GBUF * j + p
          wait_g(p)
          @pl.when(ch >= 2)
          def _():
            wait_s(p % 2)
          convert(p, p % 2)
          issue_s(ch, p % 2)
          @pl.when(ch + GBUF - 1 < NCH)
          def _():
            issue_g(ch + GBUF - 1, (p + GBUF - 1) % GBUF)
        return carry
      lax.fori_loop(0, NCH // GBUF, pipe, 0)

      wait_s(0)
      wait_s(1)
      plsc.subcore_barrier()
      ob = (c * s_half + si) * NPAD + my_rows
      pltpu.sync_copy(acc_sh.at[pl.ds(my_rows, ROWS_PER_TEC)],
                      out_hbm.at[pl.ds(ob, ROWS_PER_TEC)])

  return pl.kernel(
      body,
      out_type=jax.ShapeDtypeStruct((S * NPAD, 128), jnp.float32),
      mesh=_mesh(),
      scratch_types=[
          pltpu.VMEM((epb,), jnp.int32),
          pltpu.VMEM((epb,), jnp.int32),
          pltpu.VMEM((GBUF * CB, 64), jnp.int32),
          pltpu.VMEM((2 * CB, 128), jnp.float32),
          pltpu.VMEM_SHARED((NPAD, 128), jnp.float32),
          pltpu.SemaphoreType.DMA,
          pltpu.SemaphoreType.DMA,
          pltpu.SemaphoreType.DMA,
          pltpu.SemaphoreType.DMA,
          pltpu.SemaphoreType.DMA,
          pltpu.SemaphoreType.DMA,
      ],
      compiler_params=pltpu.CompilerParams(needs_layout_passes=False,
                                           use_tc_tiling_on_sc=False),
  )


_BN = 1000
_GRID = N // _BN


def _pack128(h128):
  words = []
  for i in range(4):
    lo = h128[:, 32 * i:32 * i + 16]
    hi = h128[:, 32 * i + 16:32 * i + 32]
    lo16 = lax.bitcast_convert_type(lo.astype(jnp.bfloat16),
                                    jnp.uint16).astype(jnp.int32)
    hi16 = lax.bitcast_convert_type(hi.astype(jnp.bfloat16),
                                    jnp.uint16).astype(jnp.int32)
    words.append((hi16 << 16) | lo16)
  return jnp.concatenate(words, axis=1)


def _mm1_body(x_ref, w_ref, d0_ref, d1_ref, hsl_ref, hslb_ref, dinv_ref):
  dinv = lax.rsqrt(1.0 + d0_ref[...] + d1_ref[...])
  dinv_ref[...] = dinv
  h = jnp.dot(x_ref[...], w_ref[...], preferred_element_type=jnp.float32)
  h = h * dinv
  for s in range(8):
    hsl_ref[s] = h[:, 128 * s:128 * (s + 1)]
    hslb_ref[s] = _pack128(h[:, 128 * s:128 * (s + 1)])


def _mm1_call(x, wcat, d0, d1):
  return pl.pallas_call(
      _mm1_body,
      grid=(_GRID,),
      in_specs=[
          pl.BlockSpec((_BN, 256), lambda i: (i, 0)),
          pl.BlockSpec((256, 1024), lambda i: (0, 0)),
          pl.BlockSpec((_BN, 1), lambda i: (i, 0)),
          pl.BlockSpec((_BN, 1), lambda i: (i, 0)),
      ],
      out_specs=[
          pl.BlockSpec((8, _BN, 128), lambda i: (0, i, 0)),
          pl.BlockSpec((8, _BN, 64), lambda i: (0, i, 0)),
          pl.BlockSpec((_BN, 1), lambda i: (i, 0)),
      ],
      out_shape=[
          jax.ShapeDtypeStruct((8, N, 128), jnp.float32),
          jax.ShapeDtypeStruct((8, N, 64), jnp.int32),
          jax.ShapeDtypeStruct((N, 1), jnp.float32),
      ],
  )(x, wcat, d0, d1)


def _ef_body(acc_ref, hsl_ref, dinv_ref, b_ref, w_ref,
             x1_ref, h1_ref, hsl2_ref, hslb2_ref):
  dinv = dinv_ref[...]
  zs = []
  for s in range(8):
    z = dinv * (acc_ref[s] + hsl_ref[s]) + b_ref[s]
    zs.append(jnp.maximum(z, 0.0))
  x1 = jnp.concatenate(zs[:4], axis=1)
  h1 = jnp.concatenate(zs[4:], axis=1)
  x1_ref[...] = x1
  h1_ref[...] = h1
  hh = jnp.dot(h1, w_ref[...], preferred_element_type=jnp.float32) * dinv
  for s in range(4):
    hsl2_ref[s] = hh[:, 128 * s:128 * (s + 1)]
    hslb2_ref[s] = _pack128(hh[:, 128 * s:128 * (s + 1)])


def _ef_call(acc1, hsl1, dinv, bcat, w2m):
  return pl.pallas_call(
      _ef_body,
      grid=(_GRID,),
      in_specs=[
          pl.BlockSpec((8, _BN, 128), lambda i: (0, i, 0)),
          pl.BlockSpec((8, _BN, 128), lambda i: (0, i, 0)),
          pl.BlockSpec((_BN, 1), lambda i: (i, 0)),
          pl.BlockSpec((8, 1, 128), lambda i: (0, 0, 0)),
          pl.BlockSpec((512, 512), lambda i: (0, 0)),
      ],
      out_specs=[
          pl.BlockSpec((_BN, 512), lambda i: (i, 0)),
          pl.BlockSpec((_BN, 512), lambda i: (i, 0)),
          pl.BlockSpec((4, _BN, 128), lambda i: (0, i, 0)),
          pl.BlockSpec((4, _BN, 64), lambda i: (0, i, 0)),
      ],
      out_shape=[
          jax.ShapeDtypeStruct((N, 512), jnp.float32),
          jax.ShapeDtypeStruct((N, 512), jnp.float32),
          jax.ShapeDtypeStruct((4, N, 128), jnp.float32),
          jax.ShapeDtypeStruct((4, N, 64), jnp.int32),
      ],
  )(acc1, hsl1, dinv, bcat, w2m)


def _gh_body(acc_ref, hsl_ref, dinv_ref, b_ref, x1_ref, wa_ref, wb_ref,
             h2_ref, hsl3_ref, hslb3_ref):
  dinv = dinv_ref[...]
  zs = []
  for s in range(4):
    z = dinv * (acc_ref[s] + hsl_ref[s]) + b_ref[s]
    zs.append(jnp.maximum(z, 0.0))
  h2 = jnp.concatenate(zs, axis=1)
  h2_ref[...] = h2
  y = (jnp.dot(x1_ref[...], wa_ref[...], preferred_element_type=jnp.float32)
       + jnp.dot(h2, wb_ref[...], preferred_element_type=jnp.float32)) * dinv
  for s in range(2):
    hsl3_ref[s] = y[:, 128 * s:128 * (s + 1)]
    hslb3_ref[s] = _pack128(y[:, 128 * s:128 * (s + 1)])


def _gh_call(acc2, hsl2, dinv, b2, x1, wa, wb):
  return pl.pallas_call(
      _gh_body,
      grid=(_GRID,),
      in_specs=[
          pl.BlockSpec((4, _BN, 128), lambda i: (0, i, 0)),
          pl.BlockSpec((4, _BN, 128), lambda i: (0, i, 0)),
          pl.BlockSpec((_BN, 1), lambda i: (i, 0)),
          pl.BlockSpec((4, 1, 128), lambda i: (0, 0, 0)),
          pl.BlockSpec((_BN, 512), lambda i: (i, 0)),
          pl.BlockSpec((512, 256), lambda i: (0, 0)),
          pl.BlockSpec((512, 256), lambda i: (0, 0)),
      ],
      out_specs=[
          pl.BlockSpec((_BN, 512), lambda i: (i, 0)),
          pl.BlockSpec((2, _BN, 128), lambda i: (0, i, 0)),
          pl.BlockSpec((2, _BN, 64), lambda i: (0, i, 0)),
      ],
      out_shape=[
          jax.ShapeDtypeStruct((N, 512), jnp.float32),
          jax.ShapeDtypeStruct((2, N, 128), jnp.float32),
          jax.ShapeDtypeStruct((2, N, 64), jnp.int32),
      ],
  )(acc2, hsl2, dinv, b2, x1, wa, wb)


def _ep3_body(acc_ref, hsl_ref, dinv_ref, b_ref, out_ref):
  dinv = dinv_ref[...]
  for s in range(2):
    out_ref[:, 128 * s:128 * (s + 1)] = (
        dinv * (acc_ref[s] + hsl_ref[s]) + b_ref[s])


def _ep3_call(acc3, hsl3, dinv, bo):
  return pl.pallas_call(
      _ep3_body,
      grid=(_GRID,),
      in_specs=[
          pl.BlockSpec((2, _BN, 128), lambda i: (0, i, 0)),
          pl.BlockSpec((2, _BN, 128), lambda i: (0, i, 0)),
          pl.BlockSpec((_BN, 1), lambda i: (i, 0)),
          pl.BlockSpec((2, 1, 128), lambda i: (0, 0, 0)),
      ],
      out_specs=pl.BlockSpec((_BN, 256), lambda i: (i, 0)),
      out_shape=jax.ShapeDtypeStruct((N, 256), jnp.float32),
  )(acc3, hsl3, dinv, bo)


@jax.jit
def _run(x, edge_index, W1h, b1h, W2h, b2h, W2m, b2m, Wout, bout):
  src = edge_index[0]
  dst = edge_index[1]
  srcp = jnp.concatenate([src, jnp.zeros((EPAD - E,), jnp.int32)])
  dstp = jnp.concatenate([dst, jnp.full((EPAD - E,), NPAD - 1, jnp.int32)])
  dst2 = dstp
  zeros128 = jnp.zeros((ROWS_PER_TEC, 128), jnp.float32)

  degp = _deg_kernel()(dstp)
  d0 = degp[:N].reshape(N, 1)
  d1 = degp[NPAD:NPAD + N].reshape(N, 1)

  wcat = jnp.concatenate([W1h, W2h], axis=1)
  bcat = jnp.concatenate([b1h, b2h]).reshape(8, 1, 128)

  hsl1, hslb1, dinv = _mm1_call(x, wcat, d0, d1)
  acc1 = _make_prop(8)(hslb1.reshape(8 * N, 64), srcp, dst2, zeros128)
  x1, h1, hsl2, hslb2 = _ef_call(acc1.reshape(8, NPAD, 128), hsl1, dinv,
                                 bcat, W2m)

  acc2 = _make_prop(4)(hslb2.reshape(4 * N, 64), srcp, dst2, zeros128)
  h2, hsl3, hslb3 = _gh_call(acc2.reshape(4, NPAD, 128), hsl2, dinv,
                             b2m.reshape(4, 1, 128), x1, Wout[:512],
                             Wout[512:])

  acc3 = _make_prop(2)(hslb3.reshape(2 * N, 64), srcp, dst2, zeros128)
  out = _ep3_call(acc3.reshape(2, NPAD, 128), hsl3, dinv,
                  bout.reshape(2, 1, 128))
  return out, x1, h1, h2


def kernel(x, edge_index, percent, ricci_curvature,
           W1h, b1h, W2h, b2h, W2m, b2m, Wout, bout):
  del percent, ricci_curvature
  return _run(x, edge_index, W1h, b1h, W2h, b2h, W2m, b2m, Wout, bout)
